# baseline (speedup 1.0000x reference)
import functools

import jax
import jax.numpy as jnp
from jax import lax
from jax.experimental import pallas as pl
from jax.experimental.pallas import tpu as pltpu
from jax.experimental.pallas import tpu_sc as plsc

_NT = 64
_NB = 16
_EMB = 32

_BE = 16_000

_NC, _NS = 2, 16
_NW = _NC * _NS
_C = 1280


def _dg(vals, idx):
    return lax.gather(
        vals, idx[:, None],
        dimension_numbers=lax.GatherDimensionNumbers(
            offset_dims=(), collapsed_slice_dims=(0,), start_index_map=(0,)),
        slice_sizes=(1,),
        mode=lax.GatherScatterMode.PROMISE_IN_BOUNDS)


def _tc_basis_body(radialt_ref, W_ref, b_ref, out_ref):
    out_ref[...] = (
        jax.lax.dot_general(
            W_ref[...], radialt_ref[...],
            dimension_numbers=(((0,), (0,)), ((), ())),
            preferred_element_type=jnp.float32)
        + b_ref[...]
    )


def _tc_basis_t(radial_t, W, b):
    E = radial_t.shape[1]
    nblk = E // _BE
    return pl.pallas_call(
        _tc_basis_body,
        grid=(nblk,),
        in_specs=[
            pl.BlockSpec((_NB, _BE), lambda i: (0, i)),
            pl.BlockSpec((_NB, _EMB), lambda i: (0, 0)),
            pl.BlockSpec((_EMB, 1), lambda i: (0, 0)),
        ],
        out_specs=pl.BlockSpec((_EMB, _BE), lambda i: (0, i)),
        out_shape=jax.ShapeDtypeStruct((_EMB, E), jnp.float32),
    )(radial_t, W, b.reshape(_EMB, 1))


def _sc_stage(basis_t, et, ctt_flat, ntt_flat):
    E = et.shape[1]
    nchunk = E // _C
    iters = (nchunk + _NW - 1) // _NW
    mesh = plsc.VectorSubcoreMesh(core_axis_name="c", subcore_axis_name="s")

    @functools.partial(
        pl.kernel,
        out_type=jax.ShapeDtypeStruct((_EMB, E), jnp.float32),
        mesh=mesh,
        scratch_types=[
            pltpu.VMEM((2, 2, _C), jnp.int32),
            pltpu.VMEM((_NB * _NT,), jnp.float32),
            pltpu.VMEM((_NB * _NT,), jnp.float32),
            pltpu.VMEM((2, _EMB, _C), jnp.float32),
            pltpu.SemaphoreType.DMA,
            pltpu.SemaphoreType.DMA,
            pltpu.SemaphoreType.DMA,
            pltpu.SemaphoreType.DMA,
        ],
    )
    def sc(basis_hbm, et_hbm, ct_hbm, nt_hbm, out_hbm,
           et_v, ct_v, nt_v, bo_v, sem_i0, sem_i1, sem_o0, sem_o1):
        wid = lax.axis_index("s") * _NC + lax.axis_index("c")
        sem_i = (sem_i0, sem_i1)
        sem_o = (sem_o0, sem_o1)
        pltpu.sync_copy(ct_hbm, ct_v)
        pltpu.sync_copy(nt_hbm, nt_v)

        def issue_in(k, p):
            base = (wid + k * _NW) * _C
            pltpu.async_copy(
                et_hbm.at[:, pl.ds(base, _C)], et_v.at[p], sem_i[p])
            pltpu.async_copy(
                basis_hbm.at[:, pl.ds(base, _C)], bo_v.at[p], sem_i[p])

        def wait_in(p):
            pltpu.make_async_copy(
                et_hbm.at[:, pl.ds(0, _C)], et_v.at[p], sem_i[p]).wait()
            pltpu.make_async_copy(
                basis_hbm.at[:, pl.ds(0, _C)], bo_v.at[p], sem_i[p]).wait()

        def issue_out(k, p):
            base = (wid + k * _NW) * _C
            pltpu.async_copy(
                bo_v.at[p], out_hbm.at[:, pl.ds(base, _C)], sem_o[p])

        def wait_out(p):
            pltpu.make_async_copy(
                bo_v.at[p], out_hbm.at[:, pl.ds(0, _C)], sem_o[p]).wait()

        def compute(p):
            for jt in range(_EMB // 4):
                tref = ct_v if jt < 4 else nt_v
                irow = 0 if jt < 4 else 1
                banks = []
                for u in range(4):
                    jj = (jt * 4 + u) % _NB
                    banks.append(tuple(
                        tref[pl.ds(jj * _NT + 16 * kk, 16)]
                        for kk in range(4)))

                def group_body(g, c, jt=jt, banks=banks, irow=irow):
                    e0 = g * 16
                    iv = et_v[p, irow, pl.ds(e0, 16)]
                    hi = iv >> 4
                    lo = iv & 15
                    m1 = hi == 1
                    m2 = hi == 2
                    m3 = hi == 3
                    for u in range(4):
                        j = jt * 4 + u
                        b0, b1, b2, b3 = banks[u]
                        te = jnp.where(
                            m3, _dg(b3, lo),
                            jnp.where(m2, _dg(b2, lo),
                                      jnp.where(m1, _dg(b1, lo),
                                                _dg(b0, lo))))
                        bo_v[p, j, pl.ds(e0, 16)] = (
                            te * bo_v[p, j, pl.ds(e0, 16)])
                    return c

                lax.fori_loop(0, _C // 16, group_body, 0, unroll=2)

        @pl.when(wid < nchunk)
        def _():
            issue_in(0, 0)

        def pair_body(m, carry):
            for p in range(2):
                k = 2 * m + p
                cid = wid + k * _NW

                @pl.when(cid < nchunk)
                def _(k=k, p=p, m=m):
                    wait_in(p)
                    if p == 1:
                        wait_out(1 - p)
                    else:
                        @pl.when(m >= 1)
                        def _():
                            wait_out(1 - p)

                    @pl.when(wid + (k + 1) * _NW < nchunk)
                    def _(k=k, p=p):
                        issue_in(k + 1, 1 - p)

                    compute(p)
                    issue_out(k, p)

            return carry

        lax.fori_loop(0, (iters + 1) // 2, pair_body, 0)

        km = (nchunk - 1 - wid) // _NW
        for p in range(2):
            @pl.when((km & 1) == p)
            def _(p=p):
                wait_out(p)

    return sc(basis_t, et, ctt_flat, ntt_flat)


def kernel(edge_types, radial, center_table, neighbor_table, W, b):
    basis_t = _tc_basis_t(radial.T, W, b)
    et = edge_types.astype(jnp.int32)
    out_t = _sc_stage(basis_t, et,
                      center_table.T.reshape(-1), neighbor_table.T.reshape(-1))
    return out_t.T


# speedup vs baseline: 1.6408x; 1.6408x over previous
import functools

import jax
import jax.numpy as jnp
from jax import lax
from jax.experimental import pallas as pl
from jax.experimental.pallas import tpu as pltpu
from jax.experimental.pallas import tpu_sc as plsc

_NT = 64
_NB = 16
_EMB = 32

_BE = 16_000

_NC, _NS = 2, 16
_NW = _NC * _NS
_C = 1280


def _dg(vals, idx):
    return lax.gather(
        vals, idx[:, None],
        dimension_numbers=lax.GatherDimensionNumbers(
            offset_dims=(), collapsed_slice_dims=(0,), start_index_map=(0,)),
        slice_sizes=(1,),
        mode=lax.GatherScatterMode.PROMISE_IN_BOUNDS)


def _tc_basis_body(radialt_ref, W_ref, b_ref, out_ref):
    out_ref[...] = (
        jax.lax.dot_general(
            W_ref[...], radialt_ref[...],
            dimension_numbers=(((0,), (0,)), ((), ())),
            preferred_element_type=jnp.float32)
        + b_ref[...]
    )


def _tc_basis_t(radial_t, W, b):
    E = radial_t.shape[1]
    nblk = E // _BE
    return pl.pallas_call(
        _tc_basis_body,
        grid=(nblk,),
        in_specs=[
            pl.BlockSpec((_NB, _BE), lambda i: (0, i)),
            pl.BlockSpec((_NB, _EMB), lambda i: (0, 0)),
            pl.BlockSpec((_EMB, 1), lambda i: (0, 0)),
        ],
        out_specs=pl.BlockSpec((_EMB, _BE), lambda i: (0, i)),
        out_shape=jax.ShapeDtypeStruct((_EMB, E), jnp.float32),
    )(radial_t, W, b.reshape(_EMB, 1))


def _sc_stage(basis_t, et, ctt_flat, ntt_flat):
    E = et.shape[1]
    nchunk = E // _C
    iters = (nchunk + _NW - 1) // _NW
    mesh = plsc.VectorSubcoreMesh(core_axis_name="c", subcore_axis_name="s")

    @functools.partial(
        pl.kernel,
        out_type=jax.ShapeDtypeStruct((_EMB, E), jnp.float32),
        mesh=mesh,
        scratch_types=[
            pltpu.VMEM((2, 2, _C), jnp.int32),
            pltpu.VMEM((_NB * _NT,), jnp.float32),
            pltpu.VMEM((_NB * _NT,), jnp.float32),
            pltpu.VMEM((2, _EMB, _C), jnp.float32),
            pltpu.SemaphoreType.DMA,
            pltpu.SemaphoreType.DMA,
            pltpu.SemaphoreType.DMA,
            pltpu.SemaphoreType.DMA,
        ],
    )
    def sc(basis_hbm, et_hbm, ct_hbm, nt_hbm, out_hbm,
           et_v, ct_v, nt_v, bo_v, sem_i0, sem_i1, sem_o0, sem_o1):
        wid = lax.axis_index("s") * _NC + lax.axis_index("c")
        sem_i = (sem_i0, sem_i1)
        sem_o = (sem_o0, sem_o1)
        pltpu.sync_copy(ct_hbm, ct_v)
        pltpu.sync_copy(nt_hbm, nt_v)

        def issue_in(k, p):
            base = (wid + k * _NW) * _C
            pltpu.async_copy(
                et_hbm.at[:, pl.ds(base, _C)], et_v.at[p], sem_i[p])
            pltpu.async_copy(
                basis_hbm.at[:, pl.ds(base, _C)], bo_v.at[p], sem_i[p])

        def wait_in(p):
            pltpu.make_async_copy(
                et_hbm.at[:, pl.ds(0, _C)], et_v.at[p], sem_i[p]).wait()
            pltpu.make_async_copy(
                basis_hbm.at[:, pl.ds(0, _C)], bo_v.at[p], sem_i[p]).wait()

        def issue_out(k, p):
            base = (wid + k * _NW) * _C
            pltpu.async_copy(
                bo_v.at[p], out_hbm.at[:, pl.ds(base, _C)], sem_o[p])

        def wait_out(p):
            pltpu.make_async_copy(
                bo_v.at[p], out_hbm.at[:, pl.ds(0, _C)], sem_o[p]).wait()

        def compute(p):
            for jt in range(_EMB // 8):
                tref = ct_v if jt < 2 else nt_v
                irow = 0 if jt < 2 else 1
                banks = []
                for u in range(8):
                    jj = (jt * 8 + u) % _NB
                    banks.append(tuple(
                        tref[pl.ds(jj * _NT + 16 * kk, 16)]
                        for kk in range(4)))

                def group_body(g, c, jt=jt, banks=banks, irow=irow):
                    e0 = g * 16
                    iv = et_v[p, irow, pl.ds(e0, 16)]
                    hi = iv >> 4
                    lo = iv & 15
                    m1 = hi == 1
                    m2 = hi == 2
                    m3 = hi == 3
                    for u in range(8):
                        j = jt * 8 + u
                        b0, b1, b2, b3 = banks[u]
                        te = jnp.where(
                            m3, _dg(b3, lo),
                            jnp.where(m2, _dg(b2, lo),
                                      jnp.where(m1, _dg(b1, lo),
                                                _dg(b0, lo))))
                        bo_v[p, j, pl.ds(e0, 16)] = (
                            te * bo_v[p, j, pl.ds(e0, 16)])
                    return c

                lax.fori_loop(0, _C // 16, group_body, 0)

        @pl.when(wid < nchunk)
        def _():
            issue_in(0, 0)

        def pair_body(m, carry):
            for p in range(2):
                k = 2 * m + p
                cid = wid + k * _NW

                @pl.when(cid < nchunk)
                def _(k=k, p=p, m=m):
                    wait_in(p)
                    if p == 1:
                        wait_out(1 - p)
                    else:
                        @pl.when(m >= 1)
                        def _():
                            wait_out(1 - p)

                    @pl.when(wid + (k + 1) * _NW < nchunk)
                    def _(k=k, p=p):
                        issue_in(k + 1, 1 - p)

                    compute(p)
                    issue_out(k, p)

            return carry

        lax.fori_loop(0, (iters + 1) // 2, pair_body, 0)

        km = (nchunk - 1 - wid) // _NW
        for p in range(2):
            @pl.when((km & 1) == p)
            def _(p=p):
                wait_out(p)

    return sc(basis_t, et, ctt_flat, ntt_flat)


def kernel(edge_types, radial, center_table, neighbor_table, W, b):
    basis_t = _tc_basis_t(radial.T, W, b)
    et = edge_types.astype(jnp.int32)
    out_t = _sc_stage(basis_t, et,
                      center_table.T.reshape(-1), neighbor_table.T.reshape(-1))
    return out_t.T


# speedup vs baseline: 1.7882x; 1.0898x over previous
import functools

import jax
import jax.numpy as jnp
from jax import lax
from jax.experimental import pallas as pl
from jax.experimental.pallas import tpu as pltpu
from jax.experimental.pallas import tpu_sc as plsc

_NT = 64
_NB = 16
_EMB = 32

_BE = 64_000

_NC, _NS = 2, 16
_NW = _NC * _NS
_C = 1280


def _dg(vals, idx):
    return lax.gather(
        vals, idx[:, None],
        dimension_numbers=lax.GatherDimensionNumbers(
            offset_dims=(), collapsed_slice_dims=(0,), start_index_map=(0,)),
        slice_sizes=(1,),
        mode=lax.GatherScatterMode.PROMISE_IN_BOUNDS)


def _tc_basis_body(radialt_ref, W_ref, b_ref, out_ref):
    out_ref[...] = (
        jax.lax.dot_general(
            W_ref[...], radialt_ref[...],
            dimension_numbers=(((0,), (0,)), ((), ())),
            preferred_element_type=jnp.float32)
        + b_ref[...]
    )


def _tc_basis_t(radial_t, W, b):
    E = radial_t.shape[1]
    nblk = E // _BE
    return pl.pallas_call(
        _tc_basis_body,
        grid=(nblk,),
        in_specs=[
            pl.BlockSpec((_NB, _BE), lambda i: (0, i)),
            pl.BlockSpec((_NB, _EMB), lambda i: (0, 0)),
            pl.BlockSpec((_EMB, 1), lambda i: (0, 0)),
        ],
        out_specs=pl.BlockSpec((_EMB, _BE), lambda i: (0, i)),
        out_shape=jax.ShapeDtypeStruct((_EMB, E), jnp.float32),
    )(radial_t, W, b.reshape(_EMB, 1))


def _sc_stage(basis_t, et, ctt_flat, ntt_flat):
    E = et.shape[1]
    nchunk = E // _C
    iters = (nchunk + _NW - 1) // _NW
    mesh = plsc.VectorSubcoreMesh(core_axis_name="c", subcore_axis_name="s")

    @functools.partial(
        pl.kernel,
        out_type=jax.ShapeDtypeStruct((_EMB, E), jnp.float32),
        mesh=mesh,
        scratch_types=[
            pltpu.VMEM((2, 2, _C), jnp.int32),
            pltpu.VMEM((_NB * _NT,), jnp.float32),
            pltpu.VMEM((_NB * _NT,), jnp.float32),
            pltpu.VMEM((2, _EMB, _C), jnp.float32),
            pltpu.SemaphoreType.DMA,
            pltpu.SemaphoreType.DMA,
            pltpu.SemaphoreType.DMA,
            pltpu.SemaphoreType.DMA,
        ],
    )
    def sc(basis_hbm, et_hbm, ct_hbm, nt_hbm, out_hbm,
           et_v, ct_v, nt_v, bo_v, sem_i0, sem_i1, sem_o0, sem_o1):
        wid = lax.axis_index("s") * _NC + lax.axis_index("c")
        sem_i = (sem_i0, sem_i1)
        sem_o = (sem_o0, sem_o1)
        pltpu.sync_copy(ct_hbm, ct_v)
        pltpu.sync_copy(nt_hbm, nt_v)

        def issue_in(k, p):
            base = (wid + k * _NW) * _C
            pltpu.async_copy(
                et_hbm.at[:, pl.ds(base, _C)], et_v.at[p], sem_i[p])
            pltpu.async_copy(
                basis_hbm.at[:, pl.ds(base, _C)], bo_v.at[p], sem_i[p])

        def wait_in(p):
            pltpu.make_async_copy(
                et_hbm.at[:, pl.ds(0, _C)], et_v.at[p], sem_i[p]).wait()
            pltpu.make_async_copy(
                basis_hbm.at[:, pl.ds(0, _C)], bo_v.at[p], sem_i[p]).wait()

        def issue_out(k, p):
            base = (wid + k * _NW) * _C
            pltpu.async_copy(
                bo_v.at[p], out_hbm.at[:, pl.ds(base, _C)], sem_o[p])

        def wait_out(p):
            pltpu.make_async_copy(
                bo_v.at[p], out_hbm.at[:, pl.ds(0, _C)], sem_o[p]).wait()

        def compute(p):
            for jt in range(_EMB // 8):
                tref = ct_v if jt < 2 else nt_v
                irow = 0 if jt < 2 else 1
                banks = []
                for u in range(8):
                    jj = (jt * 8 + u) % _NB
                    banks.append(tuple(
                        tref[pl.ds(jj * _NT + 16 * kk, 16)]
                        for kk in range(4)))

                def group_body(g, c, jt=jt, banks=banks, irow=irow):
                    e0 = g * 16
                    iv = et_v[p, irow, pl.ds(e0, 16)]
                    hi = iv >> 4
                    lo = iv & 15
                    m1 = hi == 1
                    m2 = hi == 2
                    m3 = hi == 3
                    for u in range(8):
                        j = jt * 8 + u
                        b0, b1, b2, b3 = banks[u]
                        te = jnp.where(
                            m3, _dg(b3, lo),
                            jnp.where(m2, _dg(b2, lo),
                                      jnp.where(m1, _dg(b1, lo),
                                                _dg(b0, lo))))
                        bo_v[p, j, pl.ds(e0, 16)] = (
                            te * bo_v[p, j, pl.ds(e0, 16)])
                    return c

                lax.fori_loop(0, _C // 16, group_body, 0)

        @pl.when(wid < nchunk)
        def _():
            issue_in(0, 0)

        def pair_body(m, carry):
            for p in range(2):
                k = 2 * m + p
                cid = wid + k * _NW

                @pl.when(cid < nchunk)
                def _(k=k, p=p, m=m):
                    wait_in(p)
                    if p == 1:
                        wait_out(1 - p)
                    else:
                        @pl.when(m >= 1)
                        def _():
                            wait_out(1 - p)

                    @pl.when(wid + (k + 1) * _NW < nchunk)
                    def _(k=k, p=p):
                        issue_in(k + 1, 1 - p)

                    compute(p)
                    issue_out(k, p)

            return carry

        lax.fori_loop(0, (iters + 1) // 2, pair_body, 0)

        km = (nchunk - 1 - wid) // _NW
        for p in range(2):
            @pl.when((km & 1) == p)
            def _(p=p):
                wait_out(p)

    return sc(basis_t, et, ctt_flat, ntt_flat)


def kernel(edge_types, radial, center_table, neighbor_table, W, b):
    basis_t = _tc_basis_t(radial.T, W, b)
    et = edge_types.astype(jnp.int32)
    out_t = _sc_stage(basis_t, et,
                      center_table.T.reshape(-1), neighbor_table.T.reshape(-1))
    return out_t.T
